# baseline (speedup 1.0000x reference)
import functools

import jax
import jax.numpy as jnp
from jax import lax
from jax.experimental import pallas as pl
from jax.experimental.pallas import tpu as pltpu
from jax.experimental.pallas import tpu_sc as plsc

B = 16384
NC, NS = 2, 16
NW = NC * NS
BPW = B // NW
CH = 64
NCH = BPW // CH

ORIG_D = [64, 64, 32, 32, 16, 16, 16, 32, 32]
STD_D = [32, 64, 64, 32, 32, 16, 16, 16, 32, 32]
ALL_D = ORIG_D + STD_D
ALL_V = [100000] * 9 + [1000] * 10


def _offsets(ds):
    offs, c = [], 0
    for d in ds:
        offs.append(c)
        c += d
    return offs

ORIG_OFF = _offsets(ORIG_D)
STD_OFF = _offsets(STD_D)
D_ORIG_TOT = sum(ORIG_D)
D_STD_TOT = sum(STD_D)


def _body(*refs):
    tabs = list(refs[0:19])
    gidx = list(refs[19:38])
    sel = list(refs[38:57])
    out_o, out_s = refs[57], refs[58]
    gv, sv = refs[59], refs[60]
    P = [refs[61], refs[62]]
    comb_o, comb_s = refs[63], refs[64]
    sems = [refs[65], refs[66]]

    wid = lax.axis_index("s") * NC + lax.axis_index("c")
    base = wid * BPW

    for t in range(19):
        pltpu.sync_copy(gidx[t].at[pl.ds(base, BPW)], gv.at[pl.ds(t * BPW, BPW)])
        pltpu.sync_copy(sel[t].at[pl.ds(base, BPW)], sv.at[pl.ds(t * BPW, BPW)])

    passes = [
        (out_o, comb_o, list(range(0, 9)), ORIG_OFF, ORIG_D),
        (out_s, comb_s, list(range(9, 19)), STD_OFF, STD_D),
    ]

    for out_ref, comb, ts, c0s, ds in passes:
        nt = len(ts)

        def chunk_body(c, _, out_ref=out_ref, comb=comb, ts=ts, c0s=c0s, ds=ds, nt=nt):
            r0 = c * CH

            def fire(i):
                t = ts[i]
                return pltpu.async_copy(
                    tabs[t].at[gv.at[pl.ds(t * BPW + r0, CH)]],
                    P[i % 2],
                    sems[i % 2],
                )

            def select(i):
                t = ts[i]
                c0 = c0s[i]
                d = ds[i]
                pbuf = P[i % 2]
                for g in range(CH // 16):
                    svv = sv[pl.ds(t * BPW + r0 + g * 16, 16)]
                    for l in range(16):
                        s = svv[l]
                        r = g * 16 + l
                        for q in range(d // 16):
                            comb[r, pl.ds(c0 + q * 16, 16)] = (
                                pbuf[r, pl.ds(s + q * 16, 16)]
                            )

            dsc = fire(0)
            for i in range(nt):
                nxt = fire(i + 1) if i + 1 < nt else None
                dsc.wait()
                select(i)
                dsc = nxt
            pltpu.sync_copy(comb, out_ref.at[pl.ds(base + r0, CH), :])
            return 0

        lax.fori_loop(0, NCH, chunk_body, 0)


@jax.jit
def _run(tabs2, gidxs, sels):
    mesh = plsc.VectorSubcoreMesh(
        core_axis_name="c", subcore_axis_name="s", num_cores=NC, num_subcores=NS
    )
    fn = pl.kernel(
        _body,
        out_type=(
            jax.ShapeDtypeStruct((B, D_ORIG_TOT), jnp.float32),
            jax.ShapeDtypeStruct((B, D_STD_TOT), jnp.float32),
        ),
        mesh=mesh,
        scratch_types=(
            pltpu.VMEM((19 * BPW,), jnp.int32),
            pltpu.VMEM((19 * BPW,), jnp.int32),
            pltpu.VMEM((CH, 128), jnp.float32),
            pltpu.VMEM((CH, 128), jnp.float32),
            pltpu.VMEM((CH, D_ORIG_TOT), jnp.float32),
            pltpu.VMEM((CH, D_STD_TOT), jnp.float32),
            pltpu.SemaphoreType.DMA,
            pltpu.SemaphoreType.DMA,
        ),
    )
    return fn(*tabs2, *gidxs, *sels)


def kernel(contact_idx, W_orig_contact, bodypart_idx, W_orig_bodypart, upper_bodypart_idx, W_orig_upper_bodypart, lower_bodypart_idx, W_orig_lower_bodypart, multiple_fouls_idx, W_orig_multiple_fouls, try_to_play_idx, W_orig_try_to_play, touch_ball_idx, W_orig_touch_ball, handball_idx, W_orig_handball, handball_offence_idx, W_orig_handball_offence, offence_standard_idx, W_std_offence, contact_standard_idx, W_std_contact, bodypart_standard_idx, W_std_bodypart, upper_bodypart_standard_idx, W_std_upper_bodypart, lower_bodypart_standard_idx, W_std_lower_bodypart, multiple_fouls_standard_idx, W_std_multiple_fouls, try_to_play_standard_idx, W_std_try_to_play, touch_ball_standard_idx, W_std_touch_ball, handball_standard_idx, W_std_handball, handball_offence_standard_idx, W_std_handball_offence):
    idxs = [contact_idx, bodypart_idx, upper_bodypart_idx, lower_bodypart_idx,
            multiple_fouls_idx, try_to_play_idx, touch_ball_idx, handball_idx,
            handball_offence_idx,
            offence_standard_idx, contact_standard_idx, bodypart_standard_idx,
            upper_bodypart_standard_idx, lower_bodypart_standard_idx,
            multiple_fouls_standard_idx, try_to_play_standard_idx,
            touch_ball_standard_idx, handball_standard_idx,
            handball_offence_standard_idx]
    tabs = [W_orig_contact, W_orig_bodypart, W_orig_upper_bodypart,
            W_orig_lower_bodypart, W_orig_multiple_fouls, W_orig_try_to_play,
            W_orig_touch_ball, W_orig_handball, W_orig_handball_offence,
            W_std_offence, W_std_contact, W_std_bodypart, W_std_upper_bodypart,
            W_std_lower_bodypart, W_std_multiple_fouls, W_std_try_to_play,
            W_std_touch_ball, W_std_handball, W_std_handball_offence]
    tabs2, gidxs, sels = [], [], []
    for t in range(19):
        d = ALL_D[t]
        v = ALL_V[t]
        k = 128 // d
        tabs2.append(jnp.reshape(tabs[t], (v * d // 128, 128)))
        gidxs.append(idxs[t] // k)
        sels.append((idxs[t] % k) * d)
    return _run(tabs2, gidxs, sels)


# speedup vs baseline: 1.0841x; 1.0841x over previous
import functools

import jax
import jax.numpy as jnp
from jax import lax
from jax.experimental import pallas as pl
from jax.experimental.pallas import tpu as pltpu
from jax.experimental.pallas import tpu_sc as plsc

B = 16384
NC, NS = 2, 16
NW = NC * NS
BPW = B // NW
CH = 64
NCH = BPW // CH
NPB = 4

ORIG_D = [64, 64, 32, 32, 16, 16, 16, 32, 32]
STD_D = [32, 64, 64, 32, 32, 16, 16, 16, 32, 32]
ALL_D = ORIG_D + STD_D
ALL_V = [100000] * 9 + [1000] * 10
NT = 19


def _offsets(ds):
    offs, c = [], 0
    for d in ds:
        offs.append(c)
        c += d
    return offs

ORIG_OFF = _offsets(ORIG_D)
STD_OFF = _offsets(STD_D)
D_ORIG_TOT = sum(ORIG_D)
D_STD_TOT = sum(STD_D)
TBL = [(0, ORIG_OFF[t], ORIG_D[t]) for t in range(9)] + \
      [(1, STD_OFF[t], STD_D[t]) for t in range(10)]


def _body(*refs):
    tabs = list(refs[0:NT])
    gidx_all = refs[NT]
    sel_all = refs[NT + 1]
    out_o, out_s = refs[NT + 2], refs[NT + 3]
    gv, sv = refs[NT + 4], refs[NT + 5]
    P = list(refs[NT + 6:NT + 6 + NPB])
    combs = [refs[NT + 10], refs[NT + 11]]
    sems = list(refs[NT + 12:NT + 12 + NPB])
    osems = [refs[NT + 16], refs[NT + 17]]
    outs = [out_o, out_s]

    wid = lax.axis_index("s") * NC + lax.axis_index("c")
    base = wid * BPW

    pltpu.sync_copy(gidx_all.at[pl.ds(wid * (NT * BPW), NT * BPW)], gv)
    pltpu.sync_copy(sel_all.at[pl.ds(wid * (NT * BPW), NT * BPW)], sv)

    def chunk_body(c, _):
        r0 = c * CH

        def fire(i):
            return pltpu.async_copy(
                tabs[i].at[gv.at[pl.ds(i * BPW + r0, CH)]],
                P[i % NPB],
                sems[i % NPB],
            )

        def select(i):
            ci, c0, d = TBL[i]
            comb = combs[ci]
            pbuf = P[i % NPB]
            for g in range(CH // 16):
                svv = sv[pl.ds(i * BPW + r0 + g * 16, 16)]
                for l in range(16):
                    s = svv[l]
                    r = g * 16 + l
                    for q in range(d // 16):
                        comb[r, pl.ds(c0 + q * 16, 16)] = (
                            pbuf[r, pl.ds(s + q * 16, 16)]
                        )

        descs = [fire(i) for i in range(NPB - 1)]
        first = True
        for i in range(NT):
            if i + NPB - 1 < NT:
                descs.append(fire(i + NPB - 1))
            descs[i].wait()
            if i == 0:
                @pl.when(c > 0)
                def _():
                    for oi in range(2):
                        pltpu.make_async_copy(
                            combs[oi],
                            outs[oi].at[pl.ds(base, CH), :],
                            osems[oi],
                        ).wait()
            select(i)
        for oi in range(2):
            pltpu.async_copy(
                combs[oi], outs[oi].at[pl.ds(base + r0, CH), :], osems[oi]
            )
        return 0

    lax.fori_loop(0, NCH, chunk_body, 0)

    for oi in range(2):
        pltpu.make_async_copy(
            combs[oi], outs[oi].at[pl.ds(base, CH), :], osems[oi]
        ).wait()


@jax.jit
def _run(tabs2, gidx_all, sel_all):
    mesh = plsc.VectorSubcoreMesh(
        core_axis_name="c", subcore_axis_name="s", num_cores=NC, num_subcores=NS
    )
    fn = pl.kernel(
        _body,
        out_type=(
            jax.ShapeDtypeStruct((B, D_ORIG_TOT), jnp.float32),
            jax.ShapeDtypeStruct((B, D_STD_TOT), jnp.float32),
        ),
        mesh=mesh,
        scratch_types=(
            pltpu.VMEM((NT * BPW,), jnp.int32),
            pltpu.VMEM((NT * BPW,), jnp.int32),
            pltpu.VMEM((CH, 128), jnp.float32),
            pltpu.VMEM((CH, 128), jnp.float32),
            pltpu.VMEM((CH, 128), jnp.float32),
            pltpu.VMEM((CH, 128), jnp.float32),
            pltpu.VMEM((CH, D_ORIG_TOT), jnp.float32),
            pltpu.VMEM((CH, D_STD_TOT), jnp.float32),
            pltpu.SemaphoreType.DMA,
            pltpu.SemaphoreType.DMA,
            pltpu.SemaphoreType.DMA,
            pltpu.SemaphoreType.DMA,
            pltpu.SemaphoreType.DMA,
            pltpu.SemaphoreType.DMA,
        ),
    )
    return fn(*tabs2, gidx_all, sel_all)


def kernel(contact_idx, W_orig_contact, bodypart_idx, W_orig_bodypart, upper_bodypart_idx, W_orig_upper_bodypart, lower_bodypart_idx, W_orig_lower_bodypart, multiple_fouls_idx, W_orig_multiple_fouls, try_to_play_idx, W_orig_try_to_play, touch_ball_idx, W_orig_touch_ball, handball_idx, W_orig_handball, handball_offence_idx, W_orig_handball_offence, offence_standard_idx, W_std_offence, contact_standard_idx, W_std_contact, bodypart_standard_idx, W_std_bodypart, upper_bodypart_standard_idx, W_std_upper_bodypart, lower_bodypart_standard_idx, W_std_lower_bodypart, multiple_fouls_standard_idx, W_std_multiple_fouls, try_to_play_standard_idx, W_std_try_to_play, touch_ball_standard_idx, W_std_touch_ball, handball_standard_idx, W_std_handball, handball_offence_standard_idx, W_std_handball_offence):
    idxs = [contact_idx, bodypart_idx, upper_bodypart_idx, lower_bodypart_idx,
            multiple_fouls_idx, try_to_play_idx, touch_ball_idx, handball_idx,
            handball_offence_idx,
            offence_standard_idx, contact_standard_idx, bodypart_standard_idx,
            upper_bodypart_standard_idx, lower_bodypart_standard_idx,
            multiple_fouls_standard_idx, try_to_play_standard_idx,
            touch_ball_standard_idx, handball_standard_idx,
            handball_offence_standard_idx]
    tabs = [W_orig_contact, W_orig_bodypart, W_orig_upper_bodypart,
            W_orig_lower_bodypart, W_orig_multiple_fouls, W_orig_try_to_play,
            W_orig_touch_ball, W_orig_handball, W_orig_handball_offence,
            W_std_offence, W_std_contact, W_std_bodypart, W_std_upper_bodypart,
            W_std_lower_bodypart, W_std_multiple_fouls, W_std_try_to_play,
            W_std_touch_ball, W_std_handball, W_std_handball_offence]
    tabs2, gidxs, sels = [], [], []
    for t in range(NT):
        d = ALL_D[t]
        v = ALL_V[t]
        k = 128 // d
        tabs2.append(jnp.reshape(tabs[t], (v * d // 128, 128)))
        gidxs.append(idxs[t] // k)
        sels.append((idxs[t] % k) * d)
    gidx_all = jnp.stack(gidxs).reshape(NT, NW, BPW).transpose(1, 0, 2).reshape(-1)
    sel_all = jnp.stack(sels).reshape(NT, NW, BPW).transpose(1, 0, 2).reshape(-1)
    return _run(tabs2, gidx_all, sel_all)
